# baseline (speedup 1.0000x reference)
import jax
import jax.numpy as jnp
import numpy as np
from jax import lax
from jax.experimental import pallas as pl
from jax.experimental.pallas import tpu as pltpu
from jax.experimental.pallas import tpu_sc as plsc

NU = 50000
N = 100000
NT = 16
SL = 6256
NP = NT * SL
E = 1600000
EW = E // 32
ET = E // NT
CH1 = 10000
CH2 = 800
CH3 = 400
CH4 = 2000
CH5 = 400
B3 = 3 * 4096
BT = B3 // NT

f32 = jnp.float32
i32 = jnp.int32

_PERM = [8 * (l & 1) + (l >> 1) for l in range(16)]
_INV = [2 * (q & 7) + (q >> 3) for q in range(16)]

_SC_PARAMS = pltpu.CompilerParams(use_tc_tiling_on_sc=False)


def _iota16():
    return lax.broadcasted_iota(i32, (16,), 0)


def _k1_body(h_hbm, zeros1_hbm, ones_hbm, cnt_hbm,
             idx_v, ones_v, bb_v, cnt_sh, sem):
    c = lax.axis_index("c")
    s = lax.axis_index("s")
    pltpu.sync_copy(zeros1_hbm.at[pl.ds(0, SL)], bb_v)
    pltpu.sync_copy(bb_v, cnt_sh.at[pl.ds(s * SL, SL)])
    pltpu.sync_copy(ones_hbm, ones_v)
    plsc.subcore_barrier()
    base = (c * NT + s) * EW

    def chunk(k, _):
        pltpu.sync_copy(h_hbm.at[pl.ds(base + k * CH1, CH1)], idx_v)
        pltpu.sync_copy(ones_v, cnt_sh.at[idx_v], add=True)
        return 0

    lax.fori_loop(0, EW // CH1, chunk, 0)
    plsc.subcore_barrier()
    off = s * SL
    pltpu.sync_copy(cnt_sh.at[pl.ds(off, SL)], bb_v)
    pltpu.sync_copy(bb_v, cnt_hbm.at[pl.ds(c * NP + off, SL)])


def _k2_body(h_hbm, t_hbm, t2_hbm, dinv0_hbm, ego_stk_hbm, zeros2_hbm,
             m0_hbm,
             hv, tv, t2v, dh_v, dt_v, w_v, rows_v, db_v,
             m0_sh, dinv_sh, sem, sem2, sem3):
    c = lax.axis_index("c")
    s = lax.axis_index("s")
    for k in range(8):
        nr = 656 if k == 7 else 800
        pltpu.sync_copy(zeros2_hbm.at[pl.ds(k * 800, nr)], rows_v.at[pl.ds(0, nr)])
        pltpu.sync_copy(rows_v.at[pl.ds(0, nr)],
                        m0_sh.at[pl.ds(s * SL + k * 800, nr)])
    pltpu.sync_copy(dinv0_hbm.at[pl.ds(s * SL, SL)], db_v)
    pltpu.sync_copy(db_v, dinv_sh.at[pl.ds(s * SL, SL)])
    plsc.subcore_barrier()
    base = s * ET
    even = (_iota16() & 1) == 0

    def chunk(k, _):
        cb = base + k * CH2
        pltpu.sync_copy(h_hbm.at[pl.ds(cb, CH2)], hv)
        pltpu.sync_copy(t_hbm.at[pl.ds(cb, CH2)], tv)
        pltpu.sync_copy(t2_hbm.at[pl.ds(c * E + cb, CH2)], t2v)
        a = pltpu.async_copy(dinv_sh.at[hv], dh_v, sem)
        b = pltpu.async_copy(dinv_sh.at[tv], dt_v, sem2)
        pltpu.async_copy(ego_stk_hbm.at[t2v], rows_v, sem3).wait()
        a.wait()
        b.wait()

        def wloop(jv, _):
            b16 = 16 * jv
            w_v[pl.ds(b16, 16)] = 0.25 * dh_v[pl.ds(b16, 16)] * dt_v[pl.ds(b16, 16)]
            return 0

        lax.fori_loop(0, CH2 // 16, wloop, 0)

        def scale(j, _):
            wv = w_v[pl.ds(j, 16)]
            wrep = jnp.full((16,), wv[0], dtype=f32)
            rows_v[j, :] = rows_v[j, :] * wrep
            return 0

        lax.fori_loop(0, CH2, scale, 0)
        pltpu.sync_copy(rows_v, m0_sh.at[hv], add=True)
        return 0

    lax.fori_loop(0, ET // CH2, chunk, 0)
    plsc.subcore_barrier()
    for k in range(8):
        nr = 656 if k == 7 else 800
        off = s * SL + k * 800
        pltpu.sync_copy(m0_sh.at[pl.ds(off, nr)], rows_v.at[pl.ds(0, nr)])
        pltpu.sync_copy(rows_v.at[pl.ds(0, nr)],
                        m0_hbm.at[pl.ds(c * NP + off, nr)])


def _k3_body(h2p_hbm, t_hbm, h_stk_hbm, t_stk_hbm,
             dots_hbm,
             hv, tv, hr_v, tr_v, tb_v, dots_v, t_sh, sem, sem2):
    c = lax.axis_index("c")
    s = lax.axis_index("s")
    base = s * ET
    io = _iota16()
    half = io >> 1
    sel = [half == kk for kk in range(8)]
    for k in range(15):
        off = s * SL + k * 400
        pltpu.sync_copy(t_stk_hbm.at[pl.ds(c * N + off, 400)], tr_v)
        pltpu.sync_copy(tr_v, t_sh.at[pl.ds(off, 400)])
    off = s * SL + 6000

    @pl.when(s < 15)
    def _():
        pltpu.sync_copy(t_stk_hbm.at[pl.ds(c * N + off, 256)],
                        tr_v.at[pl.ds(0, 256)])
        pltpu.sync_copy(tr_v.at[pl.ds(0, 256)], t_sh.at[pl.ds(off, 256)])

    @pl.when(s == 15)
    def _():
        pltpu.sync_copy(t_stk_hbm.at[pl.ds(c * N + off, 160)],
                        tr_v.at[pl.ds(0, 160)])
        pltpu.sync_copy(tr_v.at[pl.ds(0, 160)], t_sh.at[pl.ds(off, 160)])

    plsc.subcore_barrier()

    def chunk(k, _):
        cb = base + k * CH3
        pltpu.sync_copy(h2p_hbm.at[pl.ds(c * E + cb, CH3)], hv)
        pltpu.sync_copy(t_hbm.at[pl.ds(cb, CH3)], tv)
        a = pltpu.async_copy(h_stk_hbm.at[hv], hr_v, sem)
        b = pltpu.async_copy(t_sh.at[tv], tr_v, sem2)
        a.wait()
        b.wait()

        def dot(j, _):
            b16 = 16 * j
            p = hr_v[j, :] * tr_v[j, :]
            tb_v[pl.ds(b16, 16)] = p
            s1 = p + tb_v[pl.ds(b16 + 8, 16)]
            tb_v[pl.ds(b16, 16)] = s1
            s2 = s1 + tb_v[pl.ds(b16 + 4, 16)]
            tb_v[pl.ds(b16, 16)] = s2
            s3 = s2 + tb_v[pl.ds(b16 + 2, 16)]
            tb_v[pl.ds(b16, 16)] = s3
            return 0

        lax.fori_loop(0, CH3, dot, 0)

        def compact(g, _):
            b128 = 128 * g
            acc = tb_v[pl.ds(b128, 16)]
            for kk in range(1, 8):
                vk = tb_v[pl.ds(b128 + 14 * kk, 16)]
                acc = jnp.where(sel[kk], vk, acc)
            dots_v[pl.ds(16 * g, 16)] = acc
            return 0

        lax.fori_loop(0, CH3 // 8, compact, 0)
        pltpu.sync_copy(dots_v, dots_hbm.at[pl.ds(c * 2 * E + 2 * cb, 2 * CH3)])
        return 0

    lax.fori_loop(0, ET // CH3, chunk, 0)


def _k4_body(h2r_hbm, dots_hbm, zeros1_hbm,
             p_hbm, deg_hbm,
             hv2, da_v, db_v, p_v, tmp_v, deg_sh, sem):
    c = lax.axis_index("c")
    s = lax.axis_index("s")
    pltpu.sync_copy(zeros1_hbm, p_v.at[pl.ds(0, 2 * SL)])
    pltpu.sync_copy(p_v.at[pl.ds(0, 2 * SL)],
                    deg_sh.at[pl.ds(s * 2 * SL, 2 * SL)])
    plsc.subcore_barrier()
    base = s * ET
    even = (_iota16() & 1) == 0
    par = _iota16() & 1

    def chunk(k, _):
        cb = base + k * CH4
        pltpu.sync_copy(h2r_hbm.at[pl.ds(2 * cb, 2 * CH4)], hv2)
        pltpu.sync_copy(dots_hbm.at[pl.ds(c * 2 * E + 2 * cb, 2 * CH4)], da_v)
        pltpu.sync_copy(dots_hbm.at[pl.ds((1 - c) * 2 * E + 2 * cb, 2 * CH4)],
                        db_v)

        def body(jv, _):
            b16 = 16 * jv
            ea = jnp.exp(da_v[pl.ds(b16, 16)])
            eb = jnp.exp(db_v[pl.ds(b16, 16)])
            tmp_v[pl.ds(8, 16)] = ea
            sa = ea + jnp.where(even, tmp_v[pl.ds(9, 16)], tmp_v[pl.ds(7, 16)])
            tmp_v[pl.ds(8, 16)] = eb
            sb = eb + jnp.where(even, tmp_v[pl.ds(9, 16)], tmp_v[pl.ds(7, 16)])
            p_v[pl.ds(b16, 16)] = ea / (sa + sb)
            hv2[pl.ds(b16, 16)] = 2 * hv2[pl.ds(b16, 16)] + par
            return 0

        lax.fori_loop(0, 2 * CH4 // 16, body, 0)
        pltpu.sync_copy(p_v, deg_sh.at[hv2], add=True)
        pltpu.sync_copy(p_v, p_hbm.at[pl.ds(c * 2 * E + 2 * cb, 2 * CH4)])
        return 0

    lax.fori_loop(0, ET // CH4, chunk, 0)
    plsc.subcore_barrier()
    for k in range(4):
        nrem = 2 * SL - 3 * 4000 if k == 3 else 4000
        off = s * 2 * SL + k * 4000
        pltpu.sync_copy(deg_sh.at[pl.ds(off, nrem)], p_v.at[pl.ds(0, nrem)])
        pltpu.sync_copy(p_v.at[pl.ds(0, nrem)],
                        deg_hbm.at[pl.ds(c * 2 * NP + off, nrem)])


def _k5_body(h_hbm, t2_hbm, h2r_hbm, t2r_hbm, p_hbm, dinv1_hbm,
             ego_stk_hbm, zeros2_hbm,
             m1_hbm,
             hv, t2v, h2, t2r, pp_v, dh_v, dt_v, w_v, rows_v, db_v,
             m1_sh, dinv_sh, sem, sem2, sem3):
    c = lax.axis_index("c")
    s = lax.axis_index("s")
    for k in range(16):
        nr = 256 if k == 15 else 400
        pltpu.sync_copy(zeros2_hbm.at[pl.ds(k * 400, nr)], rows_v.at[pl.ds(0, nr)])
        pltpu.sync_copy(rows_v.at[pl.ds(0, nr)],
                        m1_sh.at[pl.ds(s * SL + k * 400, nr)])
    for k in range(4):
        nrm = 2912 if k == 3 else 3200
        off = s * 2 * SL + k * 3200
        pltpu.sync_copy(dinv1_hbm.at[pl.ds(c * 2 * NP + off, nrm)],
                        db_v.at[pl.ds(0, nrm)])
        pltpu.sync_copy(db_v.at[pl.ds(0, nrm)], dinv_sh.at[pl.ds(off, nrm)])
    plsc.subcore_barrier()
    base = s * ET
    io = _iota16()
    even = (io & 1) == 0
    par = io & 1

    def chunk(k, _):
        cb = base + k * CH5
        pltpu.sync_copy(h_hbm.at[pl.ds(cb, CH5)], hv)
        pltpu.sync_copy(t2_hbm.at[pl.ds(c * E + cb, CH5)], t2v)
        pltpu.sync_copy(h2r_hbm.at[pl.ds(2 * cb, 2 * CH5)], h2)
        pltpu.sync_copy(t2r_hbm.at[pl.ds(2 * cb, 2 * CH5)], t2r)
        pltpu.sync_copy(p_hbm.at[pl.ds(c * 2 * E + 2 * cb, 2 * CH5)], pp_v)

        def idxloop(jv, _):
            b16 = 16 * jv
            h2[pl.ds(b16, 16)] = 2 * h2[pl.ds(b16, 16)] + par
            t2r[pl.ds(b16, 16)] = 2 * t2r[pl.ds(b16, 16)] + par
            return 0

        lax.fori_loop(0, 2 * CH5 // 16, idxloop, 0)
        a = pltpu.async_copy(dinv_sh.at[h2], dh_v, sem)
        b = pltpu.async_copy(dinv_sh.at[t2r], dt_v, sem2)
        pltpu.async_copy(ego_stk_hbm.at[t2v], rows_v, sem3).wait()
        a.wait()
        b.wait()

        def wloop(jv, _):
            b16 = 16 * jv
            w_v[pl.ds(b16, 16)] = (pp_v[pl.ds(b16, 16)]
                                   * dh_v[pl.ds(b16, 16)]
                                   * dt_v[pl.ds(b16, 16)])
            return 0

        lax.fori_loop(0, 2 * CH5 // 16, wloop, 0)

        def scale(j, _):
            wv = w_v[pl.ds(2 * j, 16)]
            wrep = jnp.where(even, jnp.full((16,), wv[0], dtype=f32),
                             jnp.full((16,), wv[1], dtype=f32))
            rows_v[j, :] = rows_v[j, :] * wrep
            return 0

        lax.fori_loop(0, CH5, scale, 0)
        pltpu.sync_copy(rows_v, m1_sh.at[hv], add=True)
        return 0

    lax.fori_loop(0, ET // CH5, chunk, 0)
    plsc.subcore_barrier()
    for k in range(16):
        nr = 256 if k == 15 else 400
        off = s * SL + k * 400
        pltpu.sync_copy(m1_sh.at[pl.ds(off, nr)], rows_v.at[pl.ds(0, nr)])
        pltpu.sync_copy(rows_v.at[pl.ds(0, nr)],
                        m1_hbm.at[pl.ds(c * NP + off, nr)])


def _k6_body(m1_hbm, ego_stk_hbm, bi2_hbm, bi2p_hbm, out_hbm,
             bi_v, bj_v, er_v, mr_v, sem, sem2):
    c = lax.axis_index("c")
    s = lax.axis_index("s")
    pltpu.sync_copy(bi2p_hbm.at[pl.ds(c * B3 + s * BT, BT)], bi_v)
    pltpu.sync_copy(bi2_hbm.at[pl.ds(c * B3 + s * BT, BT)], bj_v)
    a = pltpu.async_copy(m1_hbm.at[bi_v], mr_v, sem)
    pltpu.async_copy(ego_stk_hbm.at[bj_v], er_v, sem2).wait()
    a.wait()

    def avg(j, _):
        er_v[j, :] = 0.5 * (er_v[j, :] + mr_v[j, :])
        return 0

    lax.fori_loop(0, BT, avg, 0)
    pltpu.sync_copy(er_v, out_hbm.at[pl.ds(c * B3 + s * BT, BT)])


def _tc_dinv0_body(c_ref, o_ref):
    cnt = c_ref[0:782, :] + c_ref[782:1564, :]
    o_ref[...] = lax.rsqrt(jnp.maximum(0.25 * cnt, 1e-8))


def _tc_dinv1_body(d_ref, o_ref):
    o_ref[...] = lax.rsqrt(jnp.maximum(d_ref[...], 1e-8))


def _interleave_cols(y):
    return jnp.concatenate([y[:, p:p + 1] for p in _PERM], axis=1)


def _tc_T_body(x_ref, t_ref, e_ref):
    half = pl.program_id(0)
    x = x_ref[...]
    xc = jnp.where(half == 0, x[:, 0:16], x[:, 16:32])
    outs = []
    for k in (0, 8):
        cseg = xc[:, k:k + 8]
        nr = jnp.sqrt(jnp.sum(cseg * cseg, axis=1, keepdims=True))
        outs.append(jnp.tanh(cseg / jnp.maximum(nr, 1e-12)))
    t_ref[...] = _interleave_cols(jnp.concatenate(outs, axis=1))
    e_ref[...] = _interleave_cols(xc)


def _tc_H_body(m_ref, h_ref):
    x = m_ref[...]
    colpar = lax.broadcasted_iota(i32, x.shape, 1) & 1
    ev = (colpar == 0).astype(f32)
    sse = jnp.sum(x * x * ev, axis=1, keepdims=True)
    sso = jnp.sum(x * x * (1.0 - ev), axis=1, keepdims=True)
    nrm = jnp.where(colpar == 0, jnp.sqrt(sse), jnp.sqrt(sso))
    h_ref[...] = x / jnp.maximum(nrm, 1e-12)


def kernel(users, pos, neg, user_emb, item_emb, all_h_list, all_t_list):
    assert all_h_list.shape[0] == E and user_emb.shape == (NU, 32)
    h = all_h_list.astype(i32)
    t = all_t_list.astype(i32)
    ego = jnp.concatenate([user_emb, item_emb], axis=0)
    bidx = jnp.concatenate(
        [users.astype(i32), NU + pos.astype(i32), NU + neg.astype(i32)])
    t2 = jnp.concatenate([t, t + N])
    h2p = jnp.concatenate([h, h + NP])
    bi2 = jnp.concatenate([bidx, bidx + N])
    h2r = jnp.repeat(h, 2)
    t2r = jnp.repeat(t, 2)
    zeros1 = jnp.zeros((2 * SL,), f32)
    zeros2 = jnp.zeros((SL, 16), f32)
    ones1 = jnp.ones((CH1,), f32)

    mesh = plsc.VectorSubcoreMesh(core_axis_name="c", subcore_axis_name="s")

    k1 = pl.kernel(
        _k1_body,
        out_type=jax.ShapeDtypeStruct((2 * NP,), f32),
        mesh=mesh,
        compiler_params=_SC_PARAMS,
        scratch_types=[
            pltpu.VMEM((CH1,), i32),
            pltpu.VMEM((CH1,), f32),
            pltpu.VMEM((SL,), f32),
            pltpu.VMEM_SHARED((NP,), f32),
            pltpu.SemaphoreType.DMA,
        ],
    )
    cnt = k1(h, zeros1, ones1)

    dinv0 = pl.pallas_call(
        _tc_dinv0_body,
        out_shape=jax.ShapeDtypeStruct((782, 128), f32),
    )(cnt.reshape(1564, 128)).reshape(NP)

    T_stk, ego_stk = pl.pallas_call(
        _tc_T_body,
        grid=(2, 100),
        in_specs=[pl.BlockSpec((1000, 32), lambda c, i: (i, 0))],
        out_specs=[pl.BlockSpec((1000, 16), lambda c, i: (c * 100 + i, 0)),
                   pl.BlockSpec((1000, 16), lambda c, i: (c * 100 + i, 0))],
        out_shape=[jax.ShapeDtypeStruct((2 * N, 16), f32),
                   jax.ShapeDtypeStruct((2 * N, 16), f32)],
    )(ego)

    k2 = pl.kernel(
        _k2_body,
        out_type=jax.ShapeDtypeStruct((2 * NP, 16), f32),
        mesh=mesh,
        compiler_params=_SC_PARAMS,
        scratch_types=[
            pltpu.VMEM((CH2,), i32),
            pltpu.VMEM((CH2,), i32),
            pltpu.VMEM((CH2,), i32),
            pltpu.VMEM((CH2,), f32),
            pltpu.VMEM((CH2,), f32),
            pltpu.VMEM((CH2 + 16,), f32),
            pltpu.VMEM((CH2, 16), f32),
            pltpu.VMEM((SL,), f32),
            pltpu.VMEM_SHARED((NP, 16), f32),
            pltpu.VMEM_SHARED((NP,), f32),
            pltpu.SemaphoreType.DMA,
            pltpu.SemaphoreType.DMA,
            pltpu.SemaphoreType.DMA,
        ],
    )
    M0 = k2(h, t, t2, dinv0, ego_stk, zeros2)

    H_stk = pl.pallas_call(
        _tc_H_body,
        grid=(32,),
        in_specs=[pl.BlockSpec((SL, 16), lambda i: (i, 0))],
        out_specs=pl.BlockSpec((SL, 16), lambda i: (i, 0)),
        out_shape=jax.ShapeDtypeStruct((2 * NP, 16), f32),
    )(M0)

    k3 = pl.kernel(
        _k3_body,
        out_type=jax.ShapeDtypeStruct((4 * E,), f32),
        mesh=mesh,
        compiler_params=_SC_PARAMS,
        scratch_types=[
            pltpu.VMEM((CH3,), i32),
            pltpu.VMEM((CH3,), i32),
            pltpu.VMEM((CH3, 16), f32),
            pltpu.VMEM((CH3, 16), f32),
            pltpu.VMEM((16 * CH3 + 16,), f32),
            pltpu.VMEM((2 * CH3,), f32),
            pltpu.VMEM_SHARED((NP, 16), f32),
            pltpu.SemaphoreType.DMA,
            pltpu.SemaphoreType.DMA,
        ],
    )
    dots = k3(h2p, t, H_stk, T_stk)

    k4 = pl.kernel(
        _k4_body,
        out_type=(jax.ShapeDtypeStruct((4 * E,), f32),
                  jax.ShapeDtypeStruct((4 * NP,), f32)),
        mesh=mesh,
        compiler_params=_SC_PARAMS,
        scratch_types=[
            pltpu.VMEM((2 * CH4,), i32),
            pltpu.VMEM((2 * CH4,), f32),
            pltpu.VMEM((2 * CH4,), f32),
            pltpu.VMEM((2 * CH4,), f32),
            pltpu.VMEM((48,), f32),
            pltpu.VMEM_SHARED((2 * NP,), f32),
            pltpu.SemaphoreType.DMA,
        ],
    )
    P, deg = k4(h2r, dots, zeros1)

    dinv1 = pl.pallas_call(
        _tc_dinv1_body,
        out_shape=jax.ShapeDtypeStruct((3128, 128), f32),
    )(deg.reshape(3128, 128)).reshape(4 * NP)

    k5 = pl.kernel(
        _k5_body,
        out_type=jax.ShapeDtypeStruct((2 * NP, 16), f32),
        mesh=mesh,
        compiler_params=_SC_PARAMS,
        scratch_types=[
            pltpu.VMEM((CH5,), i32),
            pltpu.VMEM((CH5,), i32),
            pltpu.VMEM((2 * CH5,), i32),
            pltpu.VMEM((2 * CH5,), i32),
            pltpu.VMEM((2 * CH5,), f32),
            pltpu.VMEM((2 * CH5,), f32),
            pltpu.VMEM((2 * CH5,), f32),
            pltpu.VMEM((2 * CH5 + 16,), f32),
            pltpu.VMEM((CH5, 16), f32),
            pltpu.VMEM((3200,), f32),
            pltpu.VMEM_SHARED((NP, 16), f32),
            pltpu.VMEM_SHARED((2 * NP,), f32),
            pltpu.SemaphoreType.DMA,
            pltpu.SemaphoreType.DMA,
            pltpu.SemaphoreType.DMA,
        ],
    )
    M1 = k5(h, t2, h2r, t2r, P, dinv1, ego_stk, zeros2)

    bi2p = jnp.concatenate([bidx, bidx + NP])
    k6 = pl.kernel(
        _k6_body,
        out_type=jax.ShapeDtypeStruct((2 * B3, 16), f32),
        mesh=mesh,
        compiler_params=_SC_PARAMS,
        scratch_types=[
            pltpu.VMEM((BT,), i32),
            pltpu.VMEM((BT,), i32),
            pltpu.VMEM((BT, 16), f32),
            pltpu.VMEM((BT, 16), f32),
            pltpu.SemaphoreType.DMA,
            pltpu.SemaphoreType.DMA,
        ],
    )
    out_stk = k6(M1, ego_stk, bi2, bi2p)

    inv = np.array(_INV)
    o0 = out_stk[:B3][:, inv]
    o1 = out_stk[B3:][:, inv]
    out = jnp.concatenate([o0, o1], axis=1)
    return out[:4096], out[4096:8192], out[8192:]


# speedup vs baseline: 1.0669x; 1.0669x over previous
import jax
import jax.numpy as jnp
import numpy as np
from jax import lax
from jax.experimental import pallas as pl
from jax.experimental.pallas import tpu as pltpu
from jax.experimental.pallas import tpu_sc as plsc

NU = 50000
N = 100000
NT = 16
SL = 6256
NP = NT * SL
E = 1600000
EW = E // 32
ET = E // NT
CH1 = 10000
CH2 = 800
CH3 = 800
CH4 = 2000
CH5 = 800
B3 = 3 * 4096
BT = B3 // NT

f32 = jnp.float32
i32 = jnp.int32

_PERM = [8 * (l & 1) + (l >> 1) for l in range(16)]
_INV = [2 * (q & 7) + (q >> 3) for q in range(16)]

_SC_PARAMS = pltpu.CompilerParams(use_tc_tiling_on_sc=False)


def _iota16():
    return lax.broadcasted_iota(i32, (16,), 0)


def _k1_body(h_hbm, zeros1_hbm, ones_hbm, cnt_hbm,
             idx_v, ones_v, bb_v, cnt_sh, sem):
    c = lax.axis_index("c")
    s = lax.axis_index("s")
    pltpu.sync_copy(zeros1_hbm.at[pl.ds(0, SL)], bb_v)
    pltpu.sync_copy(bb_v, cnt_sh.at[pl.ds(s * SL, SL)])
    pltpu.sync_copy(ones_hbm, ones_v)
    plsc.subcore_barrier()
    base = (c * NT + s) * EW

    def chunk(k, _):
        pltpu.sync_copy(h_hbm.at[pl.ds(base + k * CH1, CH1)], idx_v)
        pltpu.sync_copy(ones_v, cnt_sh.at[idx_v], add=True)
        return 0

    lax.fori_loop(0, EW // CH1, chunk, 0)
    plsc.subcore_barrier()
    off = s * SL
    pltpu.sync_copy(cnt_sh.at[pl.ds(off, SL)], bb_v)
    pltpu.sync_copy(bb_v, cnt_hbm.at[pl.ds(c * NP + off, SL)])


def _k2_body(h_hbm, t_hbm, t2_hbm, dinv0_hbm, ego_stk_hbm, zeros2_hbm,
             m0_hbm,
             hv, tv, t2v, dh_v, dt_v, w_v, rows_v, db_v,
             m0_sh, dinv_sh, sem, sem2, sem3):
    c = lax.axis_index("c")
    s = lax.axis_index("s")
    for k in range(8):
        nr = 656 if k == 7 else 800
        pltpu.sync_copy(zeros2_hbm.at[pl.ds(k * 800, nr)], rows_v.at[pl.ds(0, nr)])
        pltpu.sync_copy(rows_v.at[pl.ds(0, nr)],
                        m0_sh.at[pl.ds(s * SL + k * 800, nr)])
    pltpu.sync_copy(dinv0_hbm.at[pl.ds(s * SL, SL)], db_v)
    pltpu.sync_copy(db_v, dinv_sh.at[pl.ds(s * SL, SL)])
    plsc.subcore_barrier()
    base = s * ET
    even = (_iota16() & 1) == 0

    def chunk(k, _):
        cb = base + k * CH2
        pltpu.sync_copy(h_hbm.at[pl.ds(cb, CH2)], hv)
        pltpu.sync_copy(t_hbm.at[pl.ds(cb, CH2)], tv)
        pltpu.sync_copy(t2_hbm.at[pl.ds(c * E + cb, CH2)], t2v)
        a = pltpu.async_copy(dinv_sh.at[hv], dh_v, sem)
        b = pltpu.async_copy(dinv_sh.at[tv], dt_v, sem2)
        pltpu.async_copy(ego_stk_hbm.at[t2v], rows_v, sem3).wait()
        a.wait()
        b.wait()

        def wloop(jv, _):
            b16 = 16 * jv
            w_v[pl.ds(b16, 16)] = 0.25 * dh_v[pl.ds(b16, 16)] * dt_v[pl.ds(b16, 16)]
            return 0

        lax.fori_loop(0, CH2 // 16, wloop, 0)

        def scale(j, _):
            wv = w_v[pl.ds(j, 16)]
            wrep = jnp.full((16,), wv[0], dtype=f32)
            rows_v[j, :] = rows_v[j, :] * wrep
            return 0

        lax.fori_loop(0, CH2, scale, 0)
        pltpu.sync_copy(rows_v, m0_sh.at[hv], add=True)
        return 0

    lax.fori_loop(0, ET // CH2, chunk, 0)
    plsc.subcore_barrier()
    for k in range(8):
        nr = 656 if k == 7 else 800
        off = s * SL + k * 800
        pltpu.sync_copy(m0_sh.at[pl.ds(off, nr)], rows_v.at[pl.ds(0, nr)])
        pltpu.sync_copy(rows_v.at[pl.ds(0, nr)],
                        m0_hbm.at[pl.ds(c * NP + off, nr)])


def _k3_body(h2p_hbm, t2_hbm, h_stk_hbm, t_stk_hbm,
             dots_hbm,
             hv0, hv1, tv0, tv1, hr0, hr1, tr0, tr1, tb_v, dots_v,
             sih0, sih1, sit0, sit1, sgh0, sgh1, sgt0, sgt1):
    c = lax.axis_index("c")
    s = lax.axis_index("s")
    base = s * ET
    io = _iota16()
    half = io >> 1
    sel = [half == kk for kk in range(8)]
    hv = [hv0, hv1]
    tv = [tv0, tv1]
    hr = [hr0, hr1]
    tr = [tr0, tr1]
    sih = [sih0, sih1]
    sit = [sit0, sit1]
    sgh = [sgh0, sgh1]
    sgt = [sgt0, sgt1]
    NC = ET // CH3

    def issue_idx(k, p):
        cb = base + k * CH3
        pltpu.async_copy(h2p_hbm.at[pl.ds(c * E + cb, CH3)], hv[p], sih[p])
        pltpu.async_copy(t2_hbm.at[pl.ds(c * E + cb, CH3)], tv[p], sit[p])

    def wait_idx_issue_gather(p):
        pltpu.make_async_copy(h2p_hbm.at[pl.ds(0, CH3)], hv[p], sih[p]).wait()
        pltpu.make_async_copy(t2_hbm.at[pl.ds(0, CH3)], tv[p], sit[p]).wait()
        pltpu.async_copy(h_stk_hbm.at[hv[p]], hr[p], sgh[p])
        pltpu.async_copy(t_stk_hbm.at[tv[p]], tr[p], sgt[p])

    def compute(k, p):
        cb = base + k * CH3
        pltpu.make_async_copy(h_stk_hbm.at[hv[p]], hr[p], sgh[p]).wait()
        pltpu.make_async_copy(t_stk_hbm.at[tv[p]], tr[p], sgt[p]).wait()

        def dot(j, _):
            b16 = 16 * j
            p_ = hr[p][j, :] * tr[p][j, :]
            tb_v[pl.ds(b16, 16)] = p_
            s1 = p_ + tb_v[pl.ds(b16 + 8, 16)]
            tb_v[pl.ds(b16, 16)] = s1
            s2 = s1 + tb_v[pl.ds(b16 + 4, 16)]
            tb_v[pl.ds(b16, 16)] = s2
            s3 = s2 + tb_v[pl.ds(b16 + 2, 16)]
            tb_v[pl.ds(b16, 16)] = s3
            return 0

        lax.fori_loop(0, CH3, dot, 0)

        def compact(g, _):
            b128 = 128 * g
            acc = tb_v[pl.ds(b128, 16)]
            for kk in range(1, 8):
                vk = tb_v[pl.ds(b128 + 14 * kk, 16)]
                acc = jnp.where(sel[kk], vk, acc)
            dots_v[pl.ds(16 * g, 16)] = acc
            return 0

        lax.fori_loop(0, CH3 // 8, compact, 0)
        pltpu.sync_copy(dots_v, dots_hbm.at[pl.ds(c * 2 * E + 2 * cb, 2 * CH3)])

    issue_idx(0, 0)
    wait_idx_issue_gather(0)

    def pair(m, _):
        k0 = 2 * m
        issue_idx(k0 + 1, 1)
        compute(k0, 0)
        wait_idx_issue_gather(1)
        issue_idx(k0 + 2, 0)
        compute(k0 + 1, 1)
        wait_idx_issue_gather(0)
        return 0

    lax.fori_loop(0, (NC - 3) // 2, pair, 0)
    issue_idx(NC - 2, 1)
    compute(NC - 3, 0)
    wait_idx_issue_gather(1)
    issue_idx(NC - 1, 0)
    compute(NC - 2, 1)
    wait_idx_issue_gather(0)
    compute(NC - 1, 0)


def _k4_body(h2r_hbm, dots_hbm, zeros1_hbm,
             p_hbm, deg_hbm,
             hv2, da_v, db_v, p_v, tmp_v, deg_sh, sem):
    c = lax.axis_index("c")
    s = lax.axis_index("s")
    pltpu.sync_copy(zeros1_hbm, p_v.at[pl.ds(0, 2 * SL)])
    pltpu.sync_copy(p_v.at[pl.ds(0, 2 * SL)],
                    deg_sh.at[pl.ds(s * 2 * SL, 2 * SL)])
    plsc.subcore_barrier()
    base = s * ET
    even = (_iota16() & 1) == 0
    par = _iota16() & 1

    def chunk(k, _):
        cb = base + k * CH4
        pltpu.sync_copy(h2r_hbm.at[pl.ds(2 * cb, 2 * CH4)], hv2)
        pltpu.sync_copy(dots_hbm.at[pl.ds(c * 2 * E + 2 * cb, 2 * CH4)], da_v)
        pltpu.sync_copy(dots_hbm.at[pl.ds((1 - c) * 2 * E + 2 * cb, 2 * CH4)],
                        db_v)

        def body(jv, _):
            b16 = 16 * jv
            ea = jnp.exp(da_v[pl.ds(b16, 16)])
            eb = jnp.exp(db_v[pl.ds(b16, 16)])
            tmp_v[pl.ds(8, 16)] = ea
            sa = ea + jnp.where(even, tmp_v[pl.ds(9, 16)], tmp_v[pl.ds(7, 16)])
            tmp_v[pl.ds(8, 16)] = eb
            sb = eb + jnp.where(even, tmp_v[pl.ds(9, 16)], tmp_v[pl.ds(7, 16)])
            p_v[pl.ds(b16, 16)] = ea / (sa + sb)
            hv2[pl.ds(b16, 16)] = 2 * hv2[pl.ds(b16, 16)] + par
            return 0

        lax.fori_loop(0, 2 * CH4 // 16, body, 0)
        pltpu.sync_copy(p_v, deg_sh.at[hv2], add=True)
        pltpu.sync_copy(p_v, p_hbm.at[pl.ds(c * 2 * E + 2 * cb, 2 * CH4)])
        return 0

    lax.fori_loop(0, ET // CH4, chunk, 0)
    plsc.subcore_barrier()
    for k in range(4):
        nrem = 2 * SL - 3 * 4000 if k == 3 else 4000
        off = s * 2 * SL + k * 4000
        pltpu.sync_copy(deg_sh.at[pl.ds(off, nrem)], p_v.at[pl.ds(0, nrem)])
        pltpu.sync_copy(p_v.at[pl.ds(0, nrem)],
                        deg_hbm.at[pl.ds(c * 2 * NP + off, nrem)])


def _k5_body(h_hbm, t2_hbm, h2r_hbm, t2r_hbm, p_hbm, dinv1_hbm,
             ego_stk_hbm, zeros2_hbm,
             m1_hbm,
             hv, t2v, h2, t2r, pp_v, dh_v, dt_v, w_v, rows_v,
             m1_sh, sem, sem2, sem3):
    c = lax.axis_index("c")
    s = lax.axis_index("s")
    for k in range(8):
        nr = 656 if k == 7 else 800
        pltpu.sync_copy(zeros2_hbm.at[pl.ds(k * 800, nr)], rows_v.at[pl.ds(0, nr)])
        pltpu.sync_copy(rows_v.at[pl.ds(0, nr)],
                        m1_sh.at[pl.ds(s * SL + k * 800, nr)])
    plsc.subcore_barrier()
    base = s * ET
    io = _iota16()
    even = (io & 1) == 0
    par = io & 1

    def chunk(k, _):
        cb = base + k * CH5
        pltpu.sync_copy(h_hbm.at[pl.ds(cb, CH5)], hv)
        pltpu.sync_copy(t2_hbm.at[pl.ds(c * E + cb, CH5)], t2v)
        pltpu.sync_copy(h2r_hbm.at[pl.ds(2 * cb, 2 * CH5)], h2)
        pltpu.sync_copy(t2r_hbm.at[pl.ds(2 * cb, 2 * CH5)], t2r)
        pltpu.sync_copy(p_hbm.at[pl.ds(c * 2 * E + 2 * cb, 2 * CH5)], pp_v)

        def idxloop(jv, _):
            b16 = 16 * jv
            coff = 2 * NP * c + par
            h2[pl.ds(b16, 16)] = 2 * h2[pl.ds(b16, 16)] + coff
            t2r[pl.ds(b16, 16)] = 2 * t2r[pl.ds(b16, 16)] + coff
            return 0

        lax.fori_loop(0, 2 * CH5 // 16, idxloop, 0)
        a = pltpu.async_copy(dinv1_hbm.at[h2], dh_v, sem)
        b = pltpu.async_copy(dinv1_hbm.at[t2r], dt_v, sem2)
        pltpu.async_copy(ego_stk_hbm.at[t2v], rows_v, sem3).wait()
        a.wait()
        b.wait()

        def wloop(jv, _):
            b16 = 16 * jv
            w_v[pl.ds(b16, 16)] = (pp_v[pl.ds(b16, 16)]
                                   * dh_v[pl.ds(b16, 16)]
                                   * dt_v[pl.ds(b16, 16)])
            return 0

        lax.fori_loop(0, 2 * CH5 // 16, wloop, 0)

        def scale(j, _):
            wv = w_v[pl.ds(2 * j, 16)]
            wrep = jnp.where(even, jnp.full((16,), wv[0], dtype=f32),
                             jnp.full((16,), wv[1], dtype=f32))
            rows_v[j, :] = rows_v[j, :] * wrep
            return 0

        lax.fori_loop(0, CH5, scale, 0)
        pltpu.sync_copy(rows_v, m1_sh.at[hv], add=True)
        return 0

    lax.fori_loop(0, ET // CH5, chunk, 0)
    plsc.subcore_barrier()
    for k in range(8):
        nr = 656 if k == 7 else 800
        off = s * SL + k * 800
        pltpu.sync_copy(m1_sh.at[pl.ds(off, nr)], rows_v.at[pl.ds(0, nr)])
        pltpu.sync_copy(rows_v.at[pl.ds(0, nr)],
                        m1_hbm.at[pl.ds(c * NP + off, nr)])


def _k6_body(m1_hbm, ego_stk_hbm, bi2_hbm, bi2p_hbm, out_hbm,
             bi_v, bj_v, er_v, mr_v, sem, sem2):
    c = lax.axis_index("c")
    s = lax.axis_index("s")
    pltpu.sync_copy(bi2p_hbm.at[pl.ds(c * B3 + s * BT, BT)], bi_v)
    pltpu.sync_copy(bi2_hbm.at[pl.ds(c * B3 + s * BT, BT)], bj_v)
    a = pltpu.async_copy(m1_hbm.at[bi_v], mr_v, sem)
    pltpu.async_copy(ego_stk_hbm.at[bj_v], er_v, sem2).wait()
    a.wait()

    def avg(j, _):
        er_v[j, :] = 0.5 * (er_v[j, :] + mr_v[j, :])
        return 0

    lax.fori_loop(0, BT, avg, 0)
    pltpu.sync_copy(er_v, out_hbm.at[pl.ds(c * B3 + s * BT, BT)])


def _tc_dinv0_body(c_ref, o_ref):
    cnt = c_ref[0:782, :] + c_ref[782:1564, :]
    o_ref[...] = lax.rsqrt(jnp.maximum(0.25 * cnt, 1e-8))


def _tc_dinv1_body(d_ref, o_ref):
    o_ref[...] = lax.rsqrt(jnp.maximum(d_ref[...], 1e-8))


def _interleave_cols(y):
    return jnp.concatenate([y[:, p:p + 1] for p in _PERM], axis=1)


def _tc_T_body(x_ref, t_ref, e_ref):
    half = pl.program_id(0)
    x = x_ref[...]
    xc = jnp.where(half == 0, x[:, 0:16], x[:, 16:32])
    outs = []
    for k in (0, 8):
        cseg = xc[:, k:k + 8]
        nr = jnp.sqrt(jnp.sum(cseg * cseg, axis=1, keepdims=True))
        outs.append(jnp.tanh(cseg / jnp.maximum(nr, 1e-12)))
    t_ref[...] = _interleave_cols(jnp.concatenate(outs, axis=1))
    e_ref[...] = _interleave_cols(xc)


def _tc_H_body(m_ref, h_ref):
    x = m_ref[...]
    colpar = lax.broadcasted_iota(i32, x.shape, 1) & 1
    ev = (colpar == 0).astype(f32)
    sse = jnp.sum(x * x * ev, axis=1, keepdims=True)
    sso = jnp.sum(x * x * (1.0 - ev), axis=1, keepdims=True)
    nrm = jnp.where(colpar == 0, jnp.sqrt(sse), jnp.sqrt(sso))
    h_ref[...] = x / jnp.maximum(nrm, 1e-12)


def kernel(users, pos, neg, user_emb, item_emb, all_h_list, all_t_list):
    assert all_h_list.shape[0] == E and user_emb.shape == (NU, 32)
    h = all_h_list.astype(i32)
    t = all_t_list.astype(i32)
    ego = jnp.concatenate([user_emb, item_emb], axis=0)
    bidx = jnp.concatenate(
        [users.astype(i32), NU + pos.astype(i32), NU + neg.astype(i32)])
    t2 = jnp.concatenate([t, t + N])
    h2p = jnp.concatenate([h, h + NP])
    bi2 = jnp.concatenate([bidx, bidx + N])
    h2r = jnp.repeat(h, 2)
    t2r = jnp.repeat(t, 2)
    zeros1 = jnp.zeros((2 * SL,), f32)
    zeros2 = jnp.zeros((SL, 16), f32)
    ones1 = jnp.ones((CH1,), f32)

    mesh = plsc.VectorSubcoreMesh(core_axis_name="c", subcore_axis_name="s")

    k1 = pl.kernel(
        _k1_body,
        out_type=jax.ShapeDtypeStruct((2 * NP,), f32),
        mesh=mesh,
        compiler_params=_SC_PARAMS,
        scratch_types=[
            pltpu.VMEM((CH1,), i32),
            pltpu.VMEM((CH1,), f32),
            pltpu.VMEM((SL,), f32),
            pltpu.VMEM_SHARED((NP,), f32),
            pltpu.SemaphoreType.DMA,
        ],
    )
    cnt = k1(h, zeros1, ones1)

    dinv0 = pl.pallas_call(
        _tc_dinv0_body,
        out_shape=jax.ShapeDtypeStruct((782, 128), f32),
    )(cnt.reshape(1564, 128)).reshape(NP)

    T_stk, ego_stk = pl.pallas_call(
        _tc_T_body,
        grid=(2, 100),
        in_specs=[pl.BlockSpec((1000, 32), lambda c, i: (i, 0))],
        out_specs=[pl.BlockSpec((1000, 16), lambda c, i: (c * 100 + i, 0)),
                   pl.BlockSpec((1000, 16), lambda c, i: (c * 100 + i, 0))],
        out_shape=[jax.ShapeDtypeStruct((2 * N, 16), f32),
                   jax.ShapeDtypeStruct((2 * N, 16), f32)],
    )(ego)

    k2 = pl.kernel(
        _k2_body,
        out_type=jax.ShapeDtypeStruct((2 * NP, 16), f32),
        mesh=mesh,
        compiler_params=_SC_PARAMS,
        scratch_types=[
            pltpu.VMEM((CH2,), i32),
            pltpu.VMEM((CH2,), i32),
            pltpu.VMEM((CH2,), i32),
            pltpu.VMEM((CH2,), f32),
            pltpu.VMEM((CH2,), f32),
            pltpu.VMEM((CH2 + 16,), f32),
            pltpu.VMEM((CH2, 16), f32),
            pltpu.VMEM((SL,), f32),
            pltpu.VMEM_SHARED((NP, 16), f32),
            pltpu.VMEM_SHARED((NP,), f32),
            pltpu.SemaphoreType.DMA,
            pltpu.SemaphoreType.DMA,
            pltpu.SemaphoreType.DMA,
        ],
    )
    M0 = k2(h, t, t2, dinv0, ego_stk, zeros2)

    H_stk = pl.pallas_call(
        _tc_H_body,
        grid=(32,),
        in_specs=[pl.BlockSpec((SL, 16), lambda i: (i, 0))],
        out_specs=pl.BlockSpec((SL, 16), lambda i: (i, 0)),
        out_shape=jax.ShapeDtypeStruct((2 * NP, 16), f32),
    )(M0)

    k3 = pl.kernel(
        _k3_body,
        out_type=jax.ShapeDtypeStruct((4 * E,), f32),
        mesh=mesh,
        compiler_params=_SC_PARAMS,
        scratch_types=(
            [pltpu.VMEM((CH3,), i32)] * 4
            + [pltpu.VMEM((CH3, 16), f32)] * 4
            + [pltpu.VMEM((16 * CH3 + 16,), f32),
               pltpu.VMEM((2 * CH3,), f32)]
            + [pltpu.SemaphoreType.DMA] * 8
        ),
    )
    dots = k3(h2p, t2, H_stk, T_stk)

    k4 = pl.kernel(
        _k4_body,
        out_type=(jax.ShapeDtypeStruct((4 * E,), f32),
                  jax.ShapeDtypeStruct((4 * NP,), f32)),
        mesh=mesh,
        compiler_params=_SC_PARAMS,
        scratch_types=[
            pltpu.VMEM((2 * CH4,), i32),
            pltpu.VMEM((2 * CH4,), f32),
            pltpu.VMEM((2 * CH4,), f32),
            pltpu.VMEM((2 * CH4,), f32),
            pltpu.VMEM((48,), f32),
            pltpu.VMEM_SHARED((2 * NP,), f32),
            pltpu.SemaphoreType.DMA,
        ],
    )
    P, deg = k4(h2r, dots, zeros1)

    dinv1 = pl.pallas_call(
        _tc_dinv1_body,
        out_shape=jax.ShapeDtypeStruct((3128, 128), f32),
    )(deg.reshape(3128, 128)).reshape(4 * NP)

    k5 = pl.kernel(
        _k5_body,
        out_type=jax.ShapeDtypeStruct((2 * NP, 16), f32),
        mesh=mesh,
        compiler_params=_SC_PARAMS,
        scratch_types=[
            pltpu.VMEM((CH5,), i32),
            pltpu.VMEM((CH5,), i32),
            pltpu.VMEM((2 * CH5,), i32),
            pltpu.VMEM((2 * CH5,), i32),
            pltpu.VMEM((2 * CH5,), f32),
            pltpu.VMEM((2 * CH5,), f32),
            pltpu.VMEM((2 * CH5,), f32),
            pltpu.VMEM((2 * CH5 + 16,), f32),
            pltpu.VMEM((CH5, 16), f32),
            pltpu.VMEM_SHARED((NP, 16), f32),
            pltpu.SemaphoreType.DMA,
            pltpu.SemaphoreType.DMA,
            pltpu.SemaphoreType.DMA,
        ],
    )
    M1 = k5(h, t2, h2r, t2r, P, dinv1, ego_stk, zeros2)

    bi2p = jnp.concatenate([bidx, bidx + NP])
    k6 = pl.kernel(
        _k6_body,
        out_type=jax.ShapeDtypeStruct((2 * B3, 16), f32),
        mesh=mesh,
        compiler_params=_SC_PARAMS,
        scratch_types=[
            pltpu.VMEM((BT,), i32),
            pltpu.VMEM((BT,), i32),
            pltpu.VMEM((BT, 16), f32),
            pltpu.VMEM((BT, 16), f32),
            pltpu.SemaphoreType.DMA,
            pltpu.SemaphoreType.DMA,
        ],
    )
    out_stk = k6(M1, ego_stk, bi2, bi2p)

    inv = np.array(_INV)
    o0 = out_stk[:B3][:, inv]
    o1 = out_stk[B3:][:, inv]
    out = jnp.concatenate([o0, o1], axis=1)
    return out[:4096], out[4096:8192], out[8192:]
